# Initial kernel scaffold; baseline (speedup 1.0000x reference)
import jax
import jax.numpy as jnp
from jax.experimental import pallas as pl


def kernel(assign_logits, codebooks):
    raise NotImplementedError("write your pallas kernel here")


# speedup vs baseline: 1.8807x; 1.8807x over previous
import jax
import jax.numpy as jnp
from jax.experimental import pallas as pl

_V, _D, _M, _K = 50000, 512, 4, 512
_CHUNK = _D // _M
_TV = 1000


def _dpq_tile_kernel(logits_ref, cb_ref, out_ref):
    for m in range(_M):
        x = logits_ref[:, m * _K:(m + 1) * _K]
        x = x - jnp.max(x, axis=-1, keepdims=True)
        e = jnp.exp(x)
        attn = e / jnp.sum(e, axis=-1, keepdims=True)
        out_ref[:, m * _CHUNK:(m + 1) * _CHUNK] = jnp.dot(
            attn, cb_ref[m], preferred_element_type=jnp.float32
        )


def kernel(assign_logits, codebooks):
    logits2d = assign_logits.reshape(_V, _M * _K)
    return pl.pallas_call(
        _dpq_tile_kernel,
        grid=(_V // _TV,),
        in_specs=[
            pl.BlockSpec((_TV, _M * _K), lambda i: (i, 0)),
            pl.BlockSpec((_M, _K, _CHUNK), lambda i: (0, 0, 0)),
        ],
        out_specs=pl.BlockSpec((_TV, _D), lambda i: (i, 0)),
        out_shape=jax.ShapeDtypeStruct((_V, _D), jnp.float32),
    )(logits2d, codebooks)


# speedup vs baseline: 1.9074x; 1.0142x over previous
import jax
import jax.numpy as jnp
from jax.experimental import pallas as pl

_V, _D, _M, _K = 50000, 512, 4, 512
_CHUNK = _D // _M
_TV = 2000


def _dpq_tile_kernel(logits_ref, cb_ref, out_ref):
    for m in range(_M):
        x = logits_ref[:, m * _K:(m + 1) * _K]
        x = x - jnp.max(x, axis=-1, keepdims=True)
        e = jnp.exp(x)
        attn = e / jnp.sum(e, axis=-1, keepdims=True)
        out_ref[:, m * _CHUNK:(m + 1) * _CHUNK] = jnp.dot(
            attn, cb_ref[m], preferred_element_type=jnp.float32
        )


def kernel(assign_logits, codebooks):
    logits2d = assign_logits.reshape(_V, _M * _K)
    return pl.pallas_call(
        _dpq_tile_kernel,
        grid=(_V // _TV,),
        in_specs=[
            pl.BlockSpec((_TV, _M * _K), lambda i: (i, 0)),
            pl.BlockSpec((_M, _K, _CHUNK), lambda i: (0, 0, 0)),
        ],
        out_specs=pl.BlockSpec((_TV, _D), lambda i: (i, 0)),
        out_shape=jax.ShapeDtypeStruct((_V, _D), jnp.float32),
    )(logits2d, codebooks)
